# Initial kernel scaffold; baseline (speedup 1.0000x reference)
import jax
import jax.numpy as jnp
from jax.experimental import pallas as pl


def kernel(z, weight, edge_index, edge_type):
    raise NotImplementedError("write your pallas kernel here")


# speedup vs baseline: 1.3367x; 1.3367x over previous
import jax
import jax.numpy as jnp
from jax import lax
from jax.experimental import pallas as pl
from jax.experimental.pallas import tpu as pltpu


def _round_up(x, m):
    return (x + m - 1) // m * m


def _mip_kernel(src_ref, dst_ref, et_ref, zT_ref, w_ref, o_ref):
    te = src_ref.shape[-1]
    n_nodes = zT_ref.shape[-1]
    num_et = w_ref.shape[0]

    src = src_ref[0, :]
    dst = dst_ref[0, :]
    et = et_ref[0, :]

    node_iota = lax.broadcasted_iota(jnp.int32, (n_nodes, te), 0)
    one = jnp.float32(1.0)
    zero = jnp.float32(0.0)
    oh_src = jnp.where(node_iota == src[None, :], one, zero)
    oh_dst = jnp.where(node_iota == dst[None, :], one, zero)

    zT = zT_ref[...]
    zi = jnp.dot(zT, oh_src, preferred_element_type=jnp.float32)
    zj = jnp.dot(zT, oh_dst, preferred_element_type=jnp.float32)
    u = zi * zj

    s_all = jnp.dot(w_ref[...], u, preferred_element_type=jnp.float32)
    et_iota = lax.broadcasted_iota(jnp.int32, (num_et, te), 0)
    s = jnp.sum(jnp.where(et_iota == et[None, :], s_all, zero),
                axis=0, keepdims=True)
    o_ref[...] = jax.nn.sigmoid(s)


def kernel(z, weight, edge_index, edge_type, edge_tile=2048):
    z = jnp.asarray(z)
    weight = jnp.asarray(weight)
    N, D = z.shape
    num_et = weight.shape[0]
    E = edge_index.shape[1]

    edge_tile = max(128, min(_round_up(int(edge_tile), 128), _round_up(E, 128)))
    E_pad = _round_up(E, edge_tile)
    n_tiles = E_pad // edge_tile

    src = edge_index[0].astype(jnp.int32)
    dst = edge_index[1].astype(jnp.int32)
    et = edge_type.astype(jnp.int32)
    if E_pad != E:
        pad = E_pad - E
        src = jnp.pad(src, (0, pad))
        dst = jnp.pad(dst, (0, pad))
        et = jnp.pad(et, (0, pad))

    zT = z.T.astype(jnp.float32)
    w = weight.astype(jnp.float32)
    src2 = src.reshape(1, E_pad)
    dst2 = dst.reshape(1, E_pad)
    et2 = et.reshape(1, E_pad)

    out = pl.pallas_call(
        _mip_kernel,
        out_shape=jax.ShapeDtypeStruct((1, E_pad), jnp.float32),
        grid=(n_tiles,),
        in_specs=[
            pl.BlockSpec((1, edge_tile), lambda i: (0, i)),
            pl.BlockSpec((1, edge_tile), lambda i: (0, i)),
            pl.BlockSpec((1, edge_tile), lambda i: (0, i)),
            pl.BlockSpec((D, N), lambda i: (0, 0)),
            pl.BlockSpec((num_et, D), lambda i: (0, 0)),
        ],
        out_specs=pl.BlockSpec((1, edge_tile), lambda i: (0, i)),
        compiler_params=pltpu.CompilerParams(
            dimension_semantics=("parallel",),
            vmem_limit_bytes=48 * 1024 * 1024),
    )(src2, dst2, et2, zT, w)
    return out[0, :E]


# speedup vs baseline: 1.5926x; 1.1915x over previous
import jax
import jax.numpy as jnp
from jax import lax
from jax.experimental import pallas as pl
from jax.experimental.pallas import tpu as pltpu


def _round_up(x, m):
    return (x + m - 1) // m * m


def _mip_kernel(src_ref, dst_ref, et_ref, zT_ref, w_ref, o_ref):
    te = src_ref.shape[-1]
    n_nodes = zT_ref.shape[-1]
    num_et = w_ref.shape[0]

    src = src_ref[0, :]
    dst = dst_ref[0, :]
    et = et_ref[0, :]

    node_iota = lax.broadcasted_iota(jnp.int32, (n_nodes, te), 0)
    one = jnp.float32(1.0)
    zero = jnp.float32(0.0)
    oh_src = jnp.where(node_iota == src[None, :], one, zero)
    oh_dst = jnp.where(node_iota == dst[None, :], one, zero)

    zT = zT_ref[...]
    zi = jnp.dot(zT, oh_src, preferred_element_type=jnp.float32)
    zj = jnp.dot(zT, oh_dst, preferred_element_type=jnp.float32)
    u = zi * zj

    s_all = jnp.dot(w_ref[...], u, preferred_element_type=jnp.float32)
    et_iota = lax.broadcasted_iota(jnp.int32, (num_et, te), 0)
    s = jnp.sum(jnp.where(et_iota == et[None, :], s_all, zero),
                axis=0, keepdims=True)
    o_ref[...] = jax.nn.sigmoid(s)


def kernel(z, weight, edge_index, edge_type, edge_tile=4096):
    z = jnp.asarray(z)
    weight = jnp.asarray(weight)
    N, D = z.shape
    num_et = weight.shape[0]
    E = edge_index.shape[1]

    edge_tile = max(128, min(_round_up(int(edge_tile), 128), _round_up(E, 128)))
    E_pad = _round_up(E, edge_tile)
    n_tiles = E_pad // edge_tile

    src = edge_index[0].astype(jnp.int32)
    dst = edge_index[1].astype(jnp.int32)
    et = edge_type.astype(jnp.int32)
    if E_pad != E:
        pad = E_pad - E
        src = jnp.pad(src, (0, pad))
        dst = jnp.pad(dst, (0, pad))
        et = jnp.pad(et, (0, pad))

    zT = z.T.astype(jnp.float32)
    w = weight.astype(jnp.float32)
    src2 = src.reshape(1, E_pad)
    dst2 = dst.reshape(1, E_pad)
    et2 = et.reshape(1, E_pad)

    out = pl.pallas_call(
        _mip_kernel,
        out_shape=jax.ShapeDtypeStruct((1, E_pad), jnp.float32),
        grid=(n_tiles,),
        in_specs=[
            pl.BlockSpec((1, edge_tile), lambda i: (0, i)),
            pl.BlockSpec((1, edge_tile), lambda i: (0, i)),
            pl.BlockSpec((1, edge_tile), lambda i: (0, i)),
            pl.BlockSpec((D, N), lambda i: (0, 0)),
            pl.BlockSpec((num_et, D), lambda i: (0, 0)),
        ],
        out_specs=pl.BlockSpec((1, edge_tile), lambda i: (0, i)),
        compiler_params=pltpu.CompilerParams(
            dimension_semantics=("arbitrary",),
            vmem_limit_bytes=48 * 1024 * 1024),
    )(src2, dst2, et2, zT, w)
    return out[0, :E]


# speedup vs baseline: 1.6223x; 1.0186x over previous
import jax
import jax.numpy as jnp
from jax import lax
from jax.experimental import pallas as pl
from jax.experimental.pallas import tpu as pltpu


def _round_up(x, m):
    return (x + m - 1) // m * m


def _mip_kernel(src_ref, dst_ref, et_ref, zT_ref, w_ref, o_ref):
    te = src_ref.shape[-1]
    n_nodes = zT_ref.shape[-1]
    num_et = w_ref.shape[0]

    src = src_ref[0, :]
    dst = dst_ref[0, :]
    et = et_ref[0, :]

    node_iota = lax.broadcasted_iota(jnp.int16, (n_nodes, te), 0)
    one = jnp.bfloat16(1.0)
    zero = jnp.float32(0.0)
    onebf = jnp.bfloat16(1.0)
    zerobf = jnp.bfloat16(0.0)
    src16 = src.astype(jnp.int16)
    dst16 = dst.astype(jnp.int16)
    oh_src = jnp.where(node_iota == src16[None, :], onebf, zerobf)
    oh_dst = jnp.where(node_iota == dst16[None, :], onebf, zerobf)

    zT = zT_ref[...]
    zi = jnp.dot(zT, oh_src, preferred_element_type=jnp.float32)
    zj = jnp.dot(zT, oh_dst, preferred_element_type=jnp.float32)
    u = zi * zj

    s_all = jnp.dot(w_ref[...], u, preferred_element_type=jnp.float32)
    et_iota = lax.broadcasted_iota(jnp.int32, (num_et, te), 0)
    s = jnp.sum(jnp.where(et_iota == et[None, :], s_all, zero),
                axis=0, keepdims=True)
    o_ref[...] = jax.nn.sigmoid(s)


def kernel(z, weight, edge_index, edge_type, edge_tile=4096):
    z = jnp.asarray(z)
    weight = jnp.asarray(weight)
    N, D = z.shape
    num_et = weight.shape[0]
    E = edge_index.shape[1]

    edge_tile = max(128, min(_round_up(int(edge_tile), 128), _round_up(E, 128)))
    E_pad = _round_up(E, edge_tile)
    n_tiles = E_pad // edge_tile

    src = edge_index[0].astype(jnp.int32)
    dst = edge_index[1].astype(jnp.int32)
    et = edge_type.astype(jnp.int32)
    if E_pad != E:
        pad = E_pad - E
        src = jnp.pad(src, (0, pad))
        dst = jnp.pad(dst, (0, pad))
        et = jnp.pad(et, (0, pad))

    zT = z.T.astype(jnp.float32)
    w = weight.astype(jnp.float32)
    src2 = src.reshape(1, E_pad)
    dst2 = dst.reshape(1, E_pad)
    et2 = et.reshape(1, E_pad)

    out = pl.pallas_call(
        _mip_kernel,
        out_shape=jax.ShapeDtypeStruct((1, E_pad), jnp.float32),
        grid=(n_tiles,),
        in_specs=[
            pl.BlockSpec((1, edge_tile), lambda i: (0, i)),
            pl.BlockSpec((1, edge_tile), lambda i: (0, i)),
            pl.BlockSpec((1, edge_tile), lambda i: (0, i)),
            pl.BlockSpec((D, N), lambda i: (0, 0)),
            pl.BlockSpec((num_et, D), lambda i: (0, 0)),
        ],
        out_specs=pl.BlockSpec((1, edge_tile), lambda i: (0, i)),
        compiler_params=pltpu.CompilerParams(
            dimension_semantics=("arbitrary",),
            vmem_limit_bytes=48 * 1024 * 1024),
    )(src2, dst2, et2, zT, w)
    return out[0, :E]


# speedup vs baseline: 1.8846x; 1.1617x over previous
import jax
import jax.numpy as jnp
from jax import lax
from jax.experimental import pallas as pl
from jax.experimental.pallas import tpu as pltpu


def _round_up(x, m):
    return (x + m - 1) // m * m


def _mip_kernel(src_ref, dst_ref, et_ref, L_ref, w_ref, o_ref, *, d_dim):
    te = src_ref.shape[-1]
    n_nodes = L_ref.shape[-1]
    num_et = w_ref.shape[0]

    src16 = src_ref[0, :].astype(jnp.int16)
    dst16 = dst_ref[0, :].astype(jnp.int16)
    et = et_ref[0, :]

    node_iota = lax.broadcasted_iota(jnp.int16, (n_nodes, te), 0)
    onebf = jnp.bfloat16(1.0)
    zerobf = jnp.bfloat16(0.0)
    m = (jnp.where(node_iota == src16[None, :], onebf, zerobf)
         + jnp.where(node_iota == dst16[None, :], onebf, zerobf))

    Y = jnp.dot(L_ref[...], m, preferred_element_type=jnp.float32)
    y = Y[:d_dim, :]
    A = Y[d_dim:, :]

    S = jnp.dot(w_ref[...], y * y, preferred_element_type=jnp.float32)
    C = S - A
    et_iota = lax.broadcasted_iota(jnp.int32, (num_et, te), 0)
    s = 0.5 * jnp.sum(jnp.where(et_iota == et[None, :], C, jnp.float32(0.0)),
                      axis=0, keepdims=True)
    o_ref[...] = jax.nn.sigmoid(s)


def kernel(z, weight, edge_index, edge_type, edge_tile=4096):
    import functools

    z = jnp.asarray(z)
    weight = jnp.asarray(weight)
    N, D = z.shape
    num_et = weight.shape[0]
    E = edge_index.shape[1]

    edge_tile = max(128, min(_round_up(int(edge_tile), 128), _round_up(E, 128)))
    E_pad = _round_up(E, edge_tile)
    n_tiles = E_pad // edge_tile

    src = edge_index[0].astype(jnp.int32)
    dst = edge_index[1].astype(jnp.int32)
    et = edge_type.astype(jnp.int32)
    if E_pad != E:
        pad = E_pad - E
        src = jnp.pad(src, (0, pad))
        dst = jnp.pad(dst, (0, pad))
        et = jnp.pad(et, (0, pad))

    zf = z.astype(jnp.float32)
    wf = weight.astype(jnp.float32)
    Q = jnp.dot(wf, (zf * zf).T)
    L = jnp.concatenate([zf.T, Q], axis=0)
    src2 = src.reshape(1, E_pad)
    dst2 = dst.reshape(1, E_pad)
    et2 = et.reshape(1, E_pad)

    out = pl.pallas_call(
        functools.partial(_mip_kernel, d_dim=D),
        out_shape=jax.ShapeDtypeStruct((1, E_pad), jnp.float32),
        grid=(n_tiles,),
        in_specs=[
            pl.BlockSpec((1, edge_tile), lambda i: (0, i)),
            pl.BlockSpec((1, edge_tile), lambda i: (0, i)),
            pl.BlockSpec((1, edge_tile), lambda i: (0, i)),
            pl.BlockSpec((D + num_et, N), lambda i: (0, 0)),
            pl.BlockSpec((num_et, D), lambda i: (0, 0)),
        ],
        out_specs=pl.BlockSpec((1, edge_tile), lambda i: (0, i)),
        compiler_params=pltpu.CompilerParams(
            dimension_semantics=("arbitrary",),
            vmem_limit_bytes=48 * 1024 * 1024),
    )(src2, dst2, et2, L, wf)
    return out[0, :E]


# speedup vs baseline: 2.0395x; 1.0822x over previous
import jax
import jax.numpy as jnp
from jax import lax
from jax.experimental import pallas as pl
from jax.experimental.pallas import tpu as pltpu


def _round_up(x, m):
    return (x + m - 1) // m * m


def _mip_kernel(src_ref, dst_ref, et_ref, L_ref, w_ref, o_ref, *, d_dim):
    te = src_ref.shape[-1]
    n_nodes = L_ref.shape[-1]
    num_et = w_ref.shape[0]

    src16 = src_ref[0, :].astype(jnp.int16)
    dst16 = dst_ref[0, :].astype(jnp.int16)
    et = et_ref[0, :]

    node_iota = lax.broadcasted_iota(jnp.int16, (n_nodes, te), 0)
    onebf = jnp.bfloat16(1.0)
    zerobf = jnp.bfloat16(0.0)
    m = (jnp.where(node_iota == src16[None, :], onebf, zerobf)
         + jnp.where(node_iota == dst16[None, :], onebf, zerobf))

    Y = jnp.dot(L_ref[...], m, preferred_element_type=jnp.float32)
    y = Y[:d_dim, :]
    A = Y[d_dim:, :]

    S = jnp.dot(w_ref[...], y * y, preferred_element_type=jnp.float32)
    C = S - A
    et_iota = lax.broadcasted_iota(jnp.int32, (num_et, te), 0)
    s = 0.5 * jnp.sum(jnp.where(et_iota == et[None, :], C, jnp.float32(0.0)),
                      axis=0, keepdims=True)
    o_ref[...] = jax.nn.sigmoid(s)


def kernel(z, weight, edge_index, edge_type, edge_tile=8192):
    import functools

    z = jnp.asarray(z)
    weight = jnp.asarray(weight)
    N, D = z.shape
    num_et = weight.shape[0]
    E = edge_index.shape[1]

    edge_tile = max(128, min(_round_up(int(edge_tile), 128), _round_up(E, 128)))
    E_pad = _round_up(E, edge_tile)
    n_tiles = E_pad // edge_tile

    src = edge_index[0].astype(jnp.int32)
    dst = edge_index[1].astype(jnp.int32)
    et = edge_type.astype(jnp.int32)
    if E_pad != E:
        pad = E_pad - E
        src = jnp.pad(src, (0, pad))
        dst = jnp.pad(dst, (0, pad))
        et = jnp.pad(et, (0, pad))

    zf = z.astype(jnp.float32)
    wf = weight.astype(jnp.float32)
    Q = jnp.dot(wf, (zf * zf).T)
    L = jnp.concatenate([zf.T, Q], axis=0)
    src2 = src.reshape(1, E_pad)
    dst2 = dst.reshape(1, E_pad)
    et2 = et.reshape(1, E_pad)

    out = pl.pallas_call(
        functools.partial(_mip_kernel, d_dim=D),
        out_shape=jax.ShapeDtypeStruct((1, E_pad), jnp.float32),
        grid=(n_tiles,),
        in_specs=[
            pl.BlockSpec((1, edge_tile), lambda i: (0, i)),
            pl.BlockSpec((1, edge_tile), lambda i: (0, i)),
            pl.BlockSpec((1, edge_tile), lambda i: (0, i)),
            pl.BlockSpec((D + num_et, N), lambda i: (0, 0)),
            pl.BlockSpec((num_et, D), lambda i: (0, 0)),
        ],
        out_specs=pl.BlockSpec((1, edge_tile), lambda i: (0, i)),
        compiler_params=pltpu.CompilerParams(
            dimension_semantics=("arbitrary",),
            vmem_limit_bytes=48 * 1024 * 1024),
    )(src2, dst2, et2, L, wf)
    return out[0, :E]


# speedup vs baseline: 2.0941x; 1.0267x over previous
import jax
import jax.numpy as jnp
from jax import lax
from jax.experimental import pallas as pl
from jax.experimental.pallas import tpu as pltpu


def _round_up(x, m):
    return (x + m - 1) // m * m


def _mip_kernel(src_ref, dst_ref, et_ref, L_ref, w_ref, o_ref, *, d_dim):
    te = src_ref.shape[-1]
    n_nodes = L_ref.shape[-1]
    num_et = w_ref.shape[0]

    src16 = src_ref[0, :].astype(jnp.int16)
    dst16 = dst_ref[0, :].astype(jnp.int16)
    et = et_ref[0, :]

    node_iota = lax.broadcasted_iota(jnp.int16, (n_nodes, te), 0)
    onebf = jnp.bfloat16(1.0)
    zerobf = jnp.bfloat16(0.0)
    m = (jnp.where(node_iota == src16[None, :], onebf, zerobf)
         + jnp.where(node_iota == dst16[None, :], onebf, zerobf))

    Y = jnp.dot(L_ref[...], m, preferred_element_type=jnp.float32)
    y = Y[:d_dim, :]
    A = Y[d_dim:, :]

    S = jnp.dot(w_ref[...], y * y, preferred_element_type=jnp.float32)
    C = S - A
    et_iota = lax.broadcasted_iota(jnp.int32, (num_et, te), 0)
    s = 0.5 * jnp.sum(jnp.where(et_iota == et[None, :], C, jnp.float32(0.0)),
                      axis=0, keepdims=True)
    o_ref[...] = jax.nn.sigmoid(s)


def kernel(z, weight, edge_index, edge_type, edge_tile=16384):
    import functools

    z = jnp.asarray(z)
    weight = jnp.asarray(weight)
    N, D = z.shape
    num_et = weight.shape[0]
    E = edge_index.shape[1]

    edge_tile = max(128, min(_round_up(int(edge_tile), 128), _round_up(E, 128)))
    E_pad = _round_up(E, edge_tile)
    n_tiles = E_pad // edge_tile

    src = edge_index[0].astype(jnp.int32)
    dst = edge_index[1].astype(jnp.int32)
    et = edge_type.astype(jnp.int32)
    if E_pad != E:
        pad = E_pad - E
        src = jnp.pad(src, (0, pad))
        dst = jnp.pad(dst, (0, pad))
        et = jnp.pad(et, (0, pad))

    zf = z.astype(jnp.float32)
    wf = weight.astype(jnp.float32)
    Q = jnp.dot(wf, (zf * zf).T)
    L = jnp.concatenate([zf.T, Q], axis=0)
    src2 = src.reshape(1, E_pad)
    dst2 = dst.reshape(1, E_pad)
    et2 = et.reshape(1, E_pad)

    out = pl.pallas_call(
        functools.partial(_mip_kernel, d_dim=D),
        out_shape=jax.ShapeDtypeStruct((1, E_pad), jnp.float32),
        grid=(n_tiles,),
        in_specs=[
            pl.BlockSpec((1, edge_tile), lambda i: (0, i)),
            pl.BlockSpec((1, edge_tile), lambda i: (0, i)),
            pl.BlockSpec((1, edge_tile), lambda i: (0, i)),
            pl.BlockSpec((D + num_et, N), lambda i: (0, 0)),
            pl.BlockSpec((num_et, D), lambda i: (0, 0)),
        ],
        out_specs=pl.BlockSpec((1, edge_tile), lambda i: (0, i)),
        compiler_params=pltpu.CompilerParams(
            dimension_semantics=("arbitrary",),
            vmem_limit_bytes=48 * 1024 * 1024),
    )(src2, dst2, et2, L, wf)
    return out[0, :E]


# speedup vs baseline: 2.1671x; 1.0349x over previous
import jax
import jax.numpy as jnp
from jax import lax
from jax.experimental import pallas as pl
from jax.experimental.pallas import tpu as pltpu


def _round_up(x, m):
    return (x + m - 1) // m * m


def _mip_kernel(src_ref, dst_ref, et_ref, L_ref, w_ref, o_ref, *, d_dim):
    te = src_ref.shape[-1]
    n_nodes = L_ref.shape[-1]
    num_et = w_ref.shape[0]

    src16 = src_ref[0, :].astype(jnp.int16)
    dst16 = dst_ref[0, :].astype(jnp.int16)
    et = et_ref[0, :]

    node_iota = lax.broadcasted_iota(jnp.int16, (n_nodes, te), 0)
    onebf = jnp.bfloat16(1.0)
    zerobf = jnp.bfloat16(0.0)
    m = (jnp.where(node_iota == src16[None, :], onebf, zerobf)
         + jnp.where(node_iota == dst16[None, :], onebf, zerobf))

    Y = jnp.dot(L_ref[...], m, preferred_element_type=jnp.float32)
    y = Y[:d_dim, :]
    A = Y[d_dim:, :]

    y16 = y.astype(jnp.bfloat16)
    S = jnp.dot(w_ref[...], y16 * y16, preferred_element_type=jnp.float32)
    C = S - A
    et_iota = lax.broadcasted_iota(jnp.int32, (num_et, te), 0)
    s = 0.5 * jnp.sum(jnp.where(et_iota == et[None, :], C, jnp.float32(0.0)),
                      axis=0, keepdims=True)
    o_ref[...] = jax.nn.sigmoid(s)


def kernel(z, weight, edge_index, edge_type, edge_tile=16384):
    import functools

    z = jnp.asarray(z)
    weight = jnp.asarray(weight)
    N, D = z.shape
    num_et = weight.shape[0]
    E = edge_index.shape[1]

    edge_tile = max(128, min(_round_up(int(edge_tile), 128), _round_up(E, 128)))
    E_pad = _round_up(E, edge_tile)
    n_tiles = E_pad // edge_tile

    src = edge_index[0].astype(jnp.int32)
    dst = edge_index[1].astype(jnp.int32)
    et = edge_type.astype(jnp.int32)
    if E_pad != E:
        pad = E_pad - E
        src = jnp.pad(src, (0, pad))
        dst = jnp.pad(dst, (0, pad))
        et = jnp.pad(et, (0, pad))

    zb = z.astype(jnp.bfloat16)
    zbf = zb.astype(jnp.float32)
    wf = weight.astype(jnp.float32)
    Q = jnp.dot(wf, (zbf * zbf).T)
    L = jnp.concatenate([zbf.T, Q], axis=0).astype(jnp.bfloat16)
    wb = wf.astype(jnp.bfloat16)
    src2 = src.reshape(1, E_pad)
    dst2 = dst.reshape(1, E_pad)
    et2 = et.reshape(1, E_pad)

    out = pl.pallas_call(
        functools.partial(_mip_kernel, d_dim=D),
        out_shape=jax.ShapeDtypeStruct((1, E_pad), jnp.float32),
        grid=(n_tiles,),
        in_specs=[
            pl.BlockSpec((1, edge_tile), lambda i: (0, i)),
            pl.BlockSpec((1, edge_tile), lambda i: (0, i)),
            pl.BlockSpec((1, edge_tile), lambda i: (0, i)),
            pl.BlockSpec((D + num_et, N), lambda i: (0, 0)),
            pl.BlockSpec((num_et, D), lambda i: (0, 0)),
        ],
        out_specs=pl.BlockSpec((1, edge_tile), lambda i: (0, i)),
        compiler_params=pltpu.CompilerParams(
            dimension_semantics=("arbitrary",),
            vmem_limit_bytes=48 * 1024 * 1024),
    )(src2, dst2, et2, L, wb)
    return out[0, :E]


# speedup vs baseline: 2.1746x; 1.0035x over previous
import jax
import jax.numpy as jnp
from jax import lax
from jax.experimental import pallas as pl
from jax.experimental.pallas import tpu as pltpu


def _round_up(x, m):
    return (x + m - 1) // m * m


def _mip_kernel(src_ref, dst_ref, et_ref, L_ref, w_ref, o_ref, *, d_dim):
    te = src_ref.shape[-1]
    n_nodes = L_ref.shape[-1]
    num_et = w_ref.shape[0]

    src16 = src_ref[0, :].astype(jnp.int16)
    dst16 = dst_ref[0, :].astype(jnp.int16)

    node_iota = lax.broadcasted_iota(jnp.int16, (n_nodes, te), 0)
    onebf = jnp.bfloat16(1.0)
    zerobf = jnp.bfloat16(0.0)
    m = (jnp.where(node_iota == src16[None, :], onebf, zerobf)
         + jnp.where(node_iota == dst16[None, :], onebf, zerobf))

    Y = jnp.dot(L_ref[...], m, preferred_element_type=jnp.float32)
    y = Y[:d_dim, :]
    A = Y[d_dim:, :]

    y16 = y.astype(jnp.bfloat16)
    S = jnp.dot(w_ref[...], y16 * y16, preferred_element_type=jnp.float32)
    C = S - A
    et_iota = lax.broadcasted_iota(jnp.int32, (num_et, te), 0)
    c2 = jnp.sum(jnp.where(et_iota == et_ref[0, :][None, :], C, jnp.float32(0.0)),
                 axis=0, keepdims=True)
    o_ref[...] = 0.5 + 0.5 * jnp.tanh(0.25 * c2)


def kernel(z, weight, edge_index, edge_type, edge_tile=16384):
    import functools

    z = jnp.asarray(z)
    weight = jnp.asarray(weight)
    N, D = z.shape
    num_et = weight.shape[0]
    E = edge_index.shape[1]

    edge_tile = max(128, min(_round_up(int(edge_tile), 128), _round_up(E, 128)))
    E_pad = _round_up(E, edge_tile)
    n_tiles = E_pad // edge_tile

    src = edge_index[0].astype(jnp.int32)
    dst = edge_index[1].astype(jnp.int32)
    et = edge_type.astype(jnp.int32)
    if E_pad != E:
        pad = E_pad - E
        src = jnp.pad(src, (0, pad))
        dst = jnp.pad(dst, (0, pad))
        et = jnp.pad(et, (0, pad))

    zb = z.astype(jnp.bfloat16)
    zbf = zb.astype(jnp.float32)
    wf = weight.astype(jnp.float32)
    Q = jnp.dot(wf, (zbf * zbf).T)
    L = jnp.concatenate([zbf.T, Q], axis=0).astype(jnp.bfloat16)
    wb = wf.astype(jnp.bfloat16)
    src2 = src.reshape(1, E_pad)
    dst2 = dst.reshape(1, E_pad)
    et2 = et.reshape(1, E_pad)

    out = pl.pallas_call(
        functools.partial(_mip_kernel, d_dim=D),
        out_shape=jax.ShapeDtypeStruct((1, E_pad), jnp.float32),
        grid=(n_tiles,),
        in_specs=[
            pl.BlockSpec((1, edge_tile), lambda i: (0, i)),
            pl.BlockSpec((1, edge_tile), lambda i: (0, i)),
            pl.BlockSpec((1, edge_tile), lambda i: (0, i)),
            pl.BlockSpec((D + num_et, N), lambda i: (0, 0)),
            pl.BlockSpec((num_et, D), lambda i: (0, 0)),
        ],
        out_specs=pl.BlockSpec((1, edge_tile), lambda i: (0, i)),
        compiler_params=pltpu.CompilerParams(
            dimension_semantics=("arbitrary",),
            vmem_limit_bytes=48 * 1024 * 1024),
    )(src2, dst2, et2, L, wb)
    return out[0, :E]


# speedup vs baseline: 2.4345x; 1.1195x over previous
import jax
import jax.numpy as jnp
from jax import lax
from jax.experimental import pallas as pl
from jax.experimental.pallas import tpu as pltpu


def _round_up(x, m):
    return (x + m - 1) // m * m


def _mip_kernel(src_ref, dst_ref, et_ref, a_ref, L_ref, w_ref, o_ref, *, d_dim):
    te = src_ref.shape[-1]
    n_nodes = L_ref.shape[-1]
    num_et = w_ref.shape[0]

    src16 = src_ref[0, :].astype(jnp.int16)
    dst16 = dst_ref[0, :].astype(jnp.int16)

    node_iota = lax.broadcasted_iota(jnp.int16, (n_nodes, te), 0)
    onebf = jnp.bfloat16(1.0)
    zerobf = jnp.bfloat16(0.0)
    m = jnp.where(node_iota == src16[None, :], a_ref[0:1, :],
                  jnp.where(node_iota == dst16[None, :], onebf, zerobf))

    Y = jnp.dot(L_ref[...], m, preferred_element_type=jnp.float32)
    y = Y[:d_dim, :]
    A = Y[d_dim:, :]

    y16 = y.astype(jnp.bfloat16)
    S = jnp.dot(w_ref[...], y16 * y16, preferred_element_type=jnp.float32)
    C = S - A
    et_iota = lax.broadcasted_iota(jnp.int32, (num_et, te), 0)
    c2 = jnp.sum(jnp.where(et_iota == et_ref[0, :][None, :], C, jnp.float32(0.0)),
                 axis=0, keepdims=True)
    o_ref[...] = 0.5 + 0.5 * jnp.tanh(0.25 * c2)


def kernel(z, weight, edge_index, edge_type, edge_tile=16384):
    import functools

    z = jnp.asarray(z)
    weight = jnp.asarray(weight)
    N, D = z.shape
    num_et = weight.shape[0]
    E = edge_index.shape[1]

    edge_tile = max(128, min(_round_up(int(edge_tile), 128), _round_up(E, 128)))
    E_pad = _round_up(E, edge_tile)
    n_tiles = E_pad // edge_tile

    src = edge_index[0].astype(jnp.int32)
    dst = edge_index[1].astype(jnp.int32)
    et = edge_type.astype(jnp.int32)
    if E_pad != E:
        pad = E_pad - E
        src = jnp.pad(src, (0, pad))
        dst = jnp.pad(dst, (0, pad))
        et = jnp.pad(et, (0, pad))

    zb = z.astype(jnp.bfloat16)
    zbf = zb.astype(jnp.float32)
    wf = weight.astype(jnp.float32)
    Q = jnp.dot(wf, (zbf * zbf).T)
    L = jnp.concatenate([zbf.T, Q], axis=0).astype(jnp.bfloat16)
    wb = wf.astype(jnp.bfloat16)
    a_row = (1.0 + (src == dst).astype(jnp.float32)).astype(jnp.bfloat16)
    src2 = src.reshape(1, E_pad)
    dst2 = dst.reshape(1, E_pad)
    et2 = et.reshape(1, E_pad)
    a2 = a_row.reshape(1, E_pad)

    out = pl.pallas_call(
        functools.partial(_mip_kernel, d_dim=D),
        out_shape=jax.ShapeDtypeStruct((1, E_pad), jnp.float32),
        grid=(n_tiles,),
        in_specs=[
            pl.BlockSpec((1, edge_tile), lambda i: (0, i)),
            pl.BlockSpec((1, edge_tile), lambda i: (0, i)),
            pl.BlockSpec((1, edge_tile), lambda i: (0, i)),
            pl.BlockSpec((1, edge_tile), lambda i: (0, i)),
            pl.BlockSpec((D + num_et, N), lambda i: (0, 0)),
            pl.BlockSpec((num_et, D), lambda i: (0, 0)),
        ],
        out_specs=pl.BlockSpec((1, edge_tile), lambda i: (0, i)),
        compiler_params=pltpu.CompilerParams(
            dimension_semantics=("arbitrary",),
            vmem_limit_bytes=48 * 1024 * 1024),
    )(src2, dst2, et2, a2, L, wb)
    return out[0, :E]


# speedup vs baseline: 2.4707x; 1.0149x over previous
import jax
import jax.numpy as jnp
from jax import lax
from jax.experimental import pallas as pl
from jax.experimental.pallas import tpu as pltpu


def _round_up(x, m):
    return (x + m - 1) // m * m


def _mip_kernel(src_ref, dst_ref, et_ref, a_ref, L_ref, w_ref, o_ref, *, d_dim):
    te = src_ref.shape[-1]
    n_nodes = L_ref.shape[-1]
    num_et = w_ref.shape[0]

    src16 = src_ref[0, :].astype(jnp.int16)
    dst16 = dst_ref[0, :].astype(jnp.int16)

    node_iota = lax.broadcasted_iota(jnp.int16, (n_nodes, te), 0)
    onebf = jnp.bfloat16(1.0)
    zerobf = jnp.bfloat16(0.0)
    hot = (node_iota == src16[None, :]) | (node_iota == dst16[None, :])
    m = jnp.where(hot, a_ref[0:1, :], zerobf)

    Y = jnp.dot(L_ref[...], m, preferred_element_type=jnp.float32)
    y = Y[:d_dim, :]
    A = Y[d_dim:, :]

    y16 = y.astype(jnp.bfloat16)
    S = jnp.dot(w_ref[...], y16 * y16, preferred_element_type=jnp.float32)
    C = S - A
    et_iota = lax.broadcasted_iota(jnp.int32, (num_et, te), 0)
    c2 = jnp.sum(jnp.where(et_iota == et_ref[0, :][None, :], C, jnp.float32(0.0)),
                 axis=0, keepdims=True)
    o_ref[...] = 0.5 + 0.5 * jnp.tanh(0.25 * c2)


def kernel(z, weight, edge_index, edge_type, edge_tile=16384):
    import functools

    z = jnp.asarray(z)
    weight = jnp.asarray(weight)
    N, D = z.shape
    num_et = weight.shape[0]
    E = edge_index.shape[1]

    edge_tile = max(128, min(_round_up(int(edge_tile), 128), _round_up(E, 128)))
    E_pad = _round_up(E, edge_tile)
    n_tiles = E_pad // edge_tile

    src = edge_index[0].astype(jnp.int32)
    dst = edge_index[1].astype(jnp.int32)
    et = edge_type.astype(jnp.int32)
    if E_pad != E:
        pad = E_pad - E
        src = jnp.pad(src, (0, pad))
        dst = jnp.pad(dst, (0, pad))
        et = jnp.pad(et, (0, pad))

    zb = z.astype(jnp.bfloat16)
    zbf = zb.astype(jnp.float32)
    wf = weight.astype(jnp.float32)
    Q = jnp.dot(wf, (zbf * zbf).T)
    L = jnp.concatenate([zbf.T, Q], axis=0).astype(jnp.bfloat16)
    wb = wf.astype(jnp.bfloat16)
    a_row = (1.0 + (src == dst).astype(jnp.float32)).astype(jnp.bfloat16)
    src2 = src.reshape(1, E_pad)
    dst2 = dst.reshape(1, E_pad)
    et2 = et.reshape(1, E_pad)
    a2 = a_row.reshape(1, E_pad)

    out = pl.pallas_call(
        functools.partial(_mip_kernel, d_dim=D),
        out_shape=jax.ShapeDtypeStruct((1, E_pad), jnp.float32),
        grid=(n_tiles,),
        in_specs=[
            pl.BlockSpec((1, edge_tile), lambda i: (0, i)),
            pl.BlockSpec((1, edge_tile), lambda i: (0, i)),
            pl.BlockSpec((1, edge_tile), lambda i: (0, i)),
            pl.BlockSpec((1, edge_tile), lambda i: (0, i)),
            pl.BlockSpec((D + num_et, N), lambda i: (0, 0)),
            pl.BlockSpec((num_et, D), lambda i: (0, 0)),
        ],
        out_specs=pl.BlockSpec((1, edge_tile), lambda i: (0, i)),
        compiler_params=pltpu.CompilerParams(
            dimension_semantics=("arbitrary",),
            vmem_limit_bytes=48 * 1024 * 1024),
    )(src2, dst2, et2, a2, L, wb)
    return out[0, :E]
